# baseline (speedup 1.0000x reference)
import functools

import jax
import jax.numpy as jnp
import numpy as np
from jax.experimental import pallas as pl
from jax.experimental.pallas import tpu as pltpu
from jax.experimental.pallas import tpu_sc as plsc

_SCALE = 64.0
_MARGIN = float(np.radians(28.6))
_COS_M = float(np.cos(_MARGIN))
_SIN_M = float(np.sin(_MARGIN))

_BR = 16
_NST = 2
_NC = 2
_NS = 16
_SC_ROWS = 256
_CW = 512


def _sc_body(n_classes, b_per_w, lbl_hbm, x_hbm, v_hbm, part_hbm,
             lbl_v, tile_v, val_v, buf, accv, sem):
    b = _NC * _NS * b_per_w
    w = ((n_classes + 127) // 128) * 128
    wid = jax.lax.axis_index("s") * _NC + jax.lax.axis_index("c")
    base = wid * b_per_w
    lane_iota = jax.lax.iota(jnp.int32, 16)

    pltpu.sync_copy(lbl_hbm.at[pl.ds(base, b_per_w)], lbl_v)
    for g in range(b_per_w // 16):
        lchunk = lbl_v[pl.ds(g * 16, 16)]
        acc = jnp.zeros((16,), jnp.float32)
        for kk in range(16):
            k = g * 16 + kk
            lk = lchunk[kk]
            r0 = base + (k // 8) * 8
            c0 = pl.multiple_of(
                jax.lax.shift_left(jax.lax.shift_right_logical(lk, 7), 7), 128)
            pltpu.async_copy(
                x_hbm.at[pl.ds(r0, 8), pl.ds(c0, 128)], tile_v, sem).wait()
            g16 = pl.multiple_of(jax.lax.shift_left(
                jax.lax.shift_right_logical(jax.lax.bitwise_and(lk, 127), 4), 4), 16)
            chunk = tile_v[k % 8, pl.ds(g16, 16)]
            lane = jax.lax.bitwise_and(lk, 15)
            all16 = chunk.at[jax.lax.broadcast(lane, (16,))].get(
                mode="promise_in_bounds")
            acc = jnp.where(lane_iota == kk, all16, acc)
        val_v[pl.ds(g * 16, 16)] = acc
    pltpu.sync_copy(val_v, v_hbm.at[pl.ds(base, b_per_w)])

    row0 = (b - _SC_ROWS) + 8 * wid
    n_full = n_classes // _CW

    def outer(it, carry):
        cbase = pl.multiple_of(it * _CW, _CW)
        pltpu.async_copy(x_hbm.at[pl.ds(row0, 8), pl.ds(cbase, _CW)], buf,
                         sem).wait()
        out = list(carry)
        for r in range(8):
            a = out[r]
            for g in range(_CW // 16):
                a = a + jnp.exp(buf[r, pl.ds(g * 16, 16)] * _SCALE)
            out[r] = a
        return tuple(out)

    carry = tuple(jnp.zeros((16,), jnp.float32) for _ in range(8))
    carry = jax.lax.fori_loop(0, n_full, outer, carry)

    tail0 = n_full * _CW
    tail_valid = n_classes - tail0
    if tail_valid:
        tail_w = ((tail_valid + 127) // 128) * 128
        t0 = pl.multiple_of(tail0 + 0 * wid, 128)
        pltpu.async_copy(
            x_hbm.at[pl.ds(row0, 8), pl.ds(t0, tail_w)],
            buf.at[:, pl.ds(0, tail_w)], sem).wait()
        out = list(carry)
        for r in range(8):
            a = out[r]
            for g in range(tail_valid // 16):
                a = a + jnp.exp(buf[r, pl.ds(g * 16, 16)] * _SCALE)
            out[r] = a
        carry = tuple(out)

    for r in range(8):
        accv[r, pl.ds(0, 16)] = carry[r]
    pltpu.sync_copy(accv, part_hbm.at[pl.ds(wid * 8, 8)])


def _sc_gather_and_sums(logits, labels):
    b, n = logits.shape
    b_per_w = b // (_NC * _NS)
    mesh = plsc.VectorSubcoreMesh(
        core_axis_name="c", subcore_axis_name="s",
        num_cores=_NC, num_subcores=_NS)
    return pl.kernel(
        functools.partial(_sc_body, n, b_per_w),
        out_type=(jax.ShapeDtypeStruct((b,), jnp.float32),
                  jax.ShapeDtypeStruct((_SC_ROWS, 16), jnp.float32)),
        mesh=mesh,
        scratch_types=[
            pltpu.VMEM((b_per_w,), jnp.int32),
            pltpu.VMEM((8, 128), jnp.float32),
            pltpu.VMEM((b_per_w,), jnp.float32),
            pltpu.VMEM((8, _CW), jnp.float32),
            pltpu.VMEM((8, 16), jnp.float32),
            pltpu.SemaphoreType.DMA,
        ],
        compiler_params=pltpu.CompilerParams(use_tc_tiling_on_sc=True),
    )(labels, logits)


def _lane_tree(parts):
    while len(parts) > 1:
        nxt = [parts[i] + parts[i + 1] for i in range(0, len(parts) - 1, 2)]
        if len(parts) % 2:
            nxt.append(parts[-1])
        parts = nxt
    return parts[0]


def _dense_body(*refs, n_classes):
    x_refs = refs[:_NST]
    acc_refs = refs[_NST:]

    def one(x_ref, acc_ref):
        _, b, w = x_ref.shape
        x = x_ref[0]
        cols = jax.lax.broadcasted_iota(jnp.int32, (b, w), 1)
        e = jnp.where(cols < n_classes, jnp.exp(x * _SCALE), 0.0)
        sl = [e[:, k * 128:(k + 1) * 128] for k in range(w // 128)]
        acc_ref[...] = _lane_tree(sl)[None]

    for i in range(_NST):
        one(x_refs[i], acc_refs[i])


def _combine_body(*refs):
    acc_refs = refs[:_NST]
    part_ref = refs[_NST]
    v_ref = refs[_NST + 1]
    out_ref = refs[-1]
    s_tc = [jnp.sum(r[0], axis=1, keepdims=True) for r in acc_refs]
    s_sc = jnp.sum(part_ref[...], axis=1, keepdims=True)
    s_full = jnp.concatenate(s_tc + [s_sc], axis=0)
    v = v_ref[...]
    s_excl = s_full - jnp.exp(v * _SCALE)
    u0 = v * _COS_M - jnp.sqrt(jnp.maximum(1.0 - v * v, 0.0)) * _SIN_M
    bits = jax.lax.bitcast_convert_type(u0, jnp.int32)
    rnd = bits + 0x0FFF + jnp.bitwise_and(jax.lax.shift_right_logical(bits, 13), 1)
    rnd = jnp.bitwise_and(rnd, jnp.int32(~0x1FFF))
    u = jax.lax.bitcast_convert_type(rnd, jnp.float32)
    t = u * _SCALE
    logz = jnp.log(s_excl + jnp.exp(t))
    out_ref[0, 0] = jnp.mean(logz - t)


def kernel(logits, labels):
    b, n = logits.shape
    v, part = _sc_gather_and_sums(logits, labels.astype(jnp.int32))
    tc_rows = b - _SC_ROWS
    h = tc_rows // _NST
    ng = b // _BR
    x3d = logits.reshape(ng, _BR, n)
    nb = h // _BR
    w = ((n + 127) // 128) * 128
    xspec = lambda i: pl.BlockSpec((1, _BR, w), lambda j, i=i: (i * nb + j, 0, 0))
    ospec = pl.BlockSpec((1, _BR, 128), lambda j: (0, j, 0))
    oshape = jax.ShapeDtypeStruct((1, h, 128), jnp.float32)
    accs = pl.pallas_call(
        functools.partial(_dense_body, n_classes=n),
        grid=(nb,),
        in_specs=[xspec(i) for i in range(_NST)],
        out_specs=[ospec] * _NST,
        out_shape=[oshape] * _NST,
    )(*([x3d] * _NST))
    loss = pl.pallas_call(
        _combine_body,
        out_specs=pl.BlockSpec(memory_space=pltpu.SMEM),
        out_shape=jax.ShapeDtypeStruct((1, 1), jnp.float32),
    )(*accs, part, v.reshape(b, 1))
    return loss.reshape(())


# speedup vs baseline: 1.2852x; 1.2852x over previous
import functools

import jax
import jax.numpy as jnp
import numpy as np
from jax.experimental import pallas as pl
from jax.experimental.pallas import tpu as pltpu
from jax.experimental.pallas import tpu_sc as plsc

_SCALE = 64.0
_MARGIN = float(np.radians(28.6))
_COS_M = float(np.cos(_MARGIN))
_SIN_M = float(np.sin(_MARGIN))

_BR = 16
_NST = 2
_NC = 2
_NS = 16


def _sc_gather_body(b_per_w, lbl_hbm, x_hbm, out_hbm, lbl_v, tile_v, val_v, sem):
    wid = jax.lax.axis_index("s") * _NC + jax.lax.axis_index("c")
    base = wid * b_per_w
    pltpu.sync_copy(lbl_hbm.at[pl.ds(base, b_per_w)], lbl_v)
    lane_iota = jax.lax.iota(jnp.int32, 16)
    for g in range(b_per_w // 16):
        lchunk = lbl_v[pl.ds(g * 16, 16)]
        acc = jnp.zeros((16,), jnp.float32)
        for kk in range(16):
            k = g * 16 + kk
            lk = lchunk[kk]
            r0 = base + (k // 8) * 8
            c0 = pl.multiple_of(
                jax.lax.shift_left(jax.lax.shift_right_logical(lk, 7), 7), 128)
            pltpu.async_copy(
                x_hbm.at[pl.ds(r0, 8), pl.ds(c0, 128)], tile_v, sem).wait()
            g16 = pl.multiple_of(jax.lax.shift_left(
                jax.lax.shift_right_logical(jax.lax.bitwise_and(lk, 127), 4), 4), 16)
            chunk = tile_v[k % 8, pl.ds(g16, 16)]
            lane = jax.lax.bitwise_and(lk, 15)
            all16 = chunk.at[jax.lax.broadcast(lane, (16,))].get(
                mode="promise_in_bounds")
            acc = jnp.where(lane_iota == kk, all16, acc)
        val_v[pl.ds(g * 16, 16)] = acc
    pltpu.sync_copy(val_v, out_hbm.at[pl.ds(base, b_per_w)])


def _gather_label_vals(logits, labels):
    b, n = logits.shape
    b_per_w = b // (_NC * _NS)
    mesh = plsc.VectorSubcoreMesh(
        core_axis_name="c", subcore_axis_name="s",
        num_cores=_NC, num_subcores=_NS)
    return pl.kernel(
        functools.partial(_sc_gather_body, b_per_w),
        out_type=jax.ShapeDtypeStruct((b,), jnp.float32),
        mesh=mesh,
        scratch_types=[
            pltpu.VMEM((b_per_w,), jnp.int32),
            pltpu.VMEM((8, 128), jnp.float32),
            pltpu.VMEM((b_per_w,), jnp.float32),
            pltpu.SemaphoreType.DMA,
        ],
        compiler_params=pltpu.CompilerParams(use_tc_tiling_on_sc=True),
    )(labels, logits)


def _lane_tree(parts):
    while len(parts) > 1:
        nxt = [parts[i] + parts[i + 1] for i in range(0, len(parts) - 1, 2)]
        if len(parts) % 2:
            nxt.append(parts[-1])
        parts = nxt
    return parts[0]


def _dense_body(*refs, n_classes):
    x_refs = refs[:_NST]
    acc_refs = refs[_NST:]

    def one(x_ref, acc_ref):
        _, b, w = x_ref.shape
        x = x_ref[0]
        cols = jax.lax.broadcasted_iota(jnp.int32, (b, w), 1)
        e = jnp.where(cols < n_classes, jnp.exp(x * _SCALE), 0.0)
        sl = [e[:, k * 128:(k + 1) * 128] for k in range(w // 128)]
        acc_ref[...] = _lane_tree(sl)[None]

    for i in range(_NST):
        one(x_refs[i], acc_refs[i])


def _combine_body(*refs):
    acc_refs = refs[:_NST]
    v_ref = refs[_NST]
    out_ref = refs[-1]
    acc = jnp.concatenate([r[0] for r in acc_refs], axis=0)
    s_full = jnp.sum(acc, axis=1, keepdims=True)
    v = v_ref[...]
    s_excl = s_full - jnp.exp(v * _SCALE)
    u0 = v * _COS_M - jnp.sqrt(jnp.maximum(1.0 - v * v, 0.0)) * _SIN_M
    bits = jax.lax.bitcast_convert_type(u0, jnp.int32)
    rnd = bits + 0x0FFF + jnp.bitwise_and(jax.lax.shift_right_logical(bits, 13), 1)
    rnd = jnp.bitwise_and(rnd, jnp.int32(~0x1FFF))
    u = jax.lax.bitcast_convert_type(rnd, jnp.float32)
    t = u * _SCALE
    logz = jnp.log(s_excl + jnp.exp(t))
    out_ref[0, 0] = jnp.mean(logz - t)


def kernel(logits, labels):
    b, n = logits.shape
    v = _gather_label_vals(logits, labels.astype(jnp.int32))
    h = b // _NST
    x3d = logits.reshape(_NST, h, n)
    nb = h // _BR
    w = ((n + 127) // 128) * 128
    xspec = lambda i: pl.BlockSpec((1, _BR, w), lambda j, i=i: (i, j, 0))
    ospec = pl.BlockSpec((1, _BR, 128), lambda j: (0, j, 0))
    oshape = jax.ShapeDtypeStruct((1, h, 128), jnp.float32)
    accs = pl.pallas_call(
        functools.partial(_dense_body, n_classes=n),
        grid=(nb,),
        in_specs=[xspec(i) for i in range(_NST)],
        out_specs=[ospec] * _NST,
        out_shape=[oshape] * _NST,
    )(*([x3d] * _NST))
    loss = pl.pallas_call(
        _combine_body,
        out_specs=pl.BlockSpec(memory_space=pltpu.SMEM),
        out_shape=jax.ShapeDtypeStruct((1, 1), jnp.float32),
    )(*accs, v.reshape(b, 1))
    return loss.reshape(())
